# baseline (speedup 1.0000x reference)
import jax
import jax.numpy as jnp
from jax import lax
from jax.experimental import pallas as pl
from jax.experimental.pallas import tpu as pltpu
from jax.experimental.pallas import tpu_sc as plsc

_GATHER_ONLY = True

N = 10000
E = 320000
D = 128
FH = D // 2

NC = 2
NS = 16
EPT = E // NS
CHUNK = 128
NQ = 4
NCHUNK_T = 160
EPT_PAD = NCHUNK_T * CHUNK
QCHUNK = NCHUNK_T // NQ
QPAIR = QCHUNK // 2
QEDGE = QCHUNK * CHUNK
N_PAD = 10112
ROWS_PER_TILE = N_PAD // NS


def _sc_agg_body(xh_hbm, src_hbm, dst_hbm, out_hbm,
                 src_q, dst_q, rows_a, rows_b, x_sp, agg_sh, sem_a, sem_b):
    c = lax.axis_index("c")
    s = lax.axis_index("s")
    base = s * ROWS_PER_TILE

    pltpu.sync_copy(xh_hbm.at[c, pl.ds(base, ROWS_PER_TILE)],
                    x_sp.at[pl.ds(base, ROWS_PER_TILE)])

    zero16 = jnp.zeros((16,), jnp.float32)

    def zrow(r, carry):
        for k in range(FH // 16):
            rows_a[r, pl.ds(k * 16, 16)] = zero16
        return carry

    lax.fori_loop(0, CHUNK, zrow, 0)
    for t in range(ROWS_PER_TILE // CHUNK):
        pltpu.sync_copy(rows_a, agg_sh.at[pl.ds(base + t * CHUNK, CHUNK)])
    rem = ROWS_PER_TILE % CHUNK
    if rem:
        pltpu.sync_copy(
            rows_a.at[pl.ds(0, rem)],
            agg_sh.at[pl.ds(base + (ROWS_PER_TILE // CHUNK) * CHUNK, rem)],
        )

    plsc.subcore_barrier()

    for q in range(NQ):
        pltpu.sync_copy(src_hbm.at[s, q], src_q)
        pltpu.sync_copy(dst_hbm.at[s, q], dst_q)

        pltpu.async_copy(x_sp.at[src_q.at[pl.ds(0, CHUNK)]], rows_a, sem_a)

        def pair_step(i, carry):
            ja = 2 * i
            jb = 2 * i + 1
            pltpu.async_copy(x_sp.at[src_q.at[pl.ds(jb * CHUNK, CHUNK)]], rows_b, sem_b)
            pltpu.make_async_copy(x_sp.at[src_q.at[pl.ds(ja * CHUNK, CHUNK)]], rows_a, sem_a).wait()
            if not _GATHER_ONLY:
                pltpu.sync_copy(rows_a, agg_sh.at[dst_q.at[ja]], add=True)

            @pl.when(i < QPAIR - 1)
            def _():
                pltpu.async_copy(x_sp.at[src_q.at[pl.ds((ja + 2) * CHUNK, CHUNK)]], rows_a, sem_a)

            pltpu.make_async_copy(x_sp.at[src_q.at[pl.ds(jb * CHUNK, CHUNK)]], rows_b, sem_b).wait()
            if not _GATHER_ONLY:
                pltpu.sync_copy(rows_b, agg_sh.at[dst_q.at[jb]], add=True)
            return carry

        lax.fori_loop(0, QPAIR, pair_step, 0)

    plsc.subcore_barrier()

    pltpu.sync_copy(
        agg_sh.at[pl.ds(base, ROWS_PER_TILE)],
        out_hbm.at[c, pl.ds(base, ROWS_PER_TILE)],
    )


@jax.jit
def _sc_agg(xh, srcq, dstq):
    mesh = plsc.VectorSubcoreMesh(core_axis_name="c", subcore_axis_name="s")
    return pl.kernel(
        _sc_agg_body,
        out_type=jax.ShapeDtypeStruct((NC, N_PAD, FH), jnp.float32),
        mesh=mesh,
        compiler_params=pltpu.CompilerParams(use_tc_tiling_on_sc=False),
        scratch_types=[
            pltpu.VMEM((QEDGE,), jnp.int32),
            pltpu.VMEM((QCHUNK, CHUNK), jnp.int32),
            pltpu.VMEM((CHUNK, FH), jnp.float32),
            pltpu.VMEM((CHUNK, FH), jnp.float32),
            pltpu.VMEM_SHARED((N_PAD, FH), jnp.float32),
            pltpu.VMEM_SHARED((N_PAD, FH), jnp.float32),
            pltpu.SemaphoreType.DMA,
            pltpu.SemaphoreType.DMA,
        ],
    )(xh, srcq, dstq)


def _mlp_body(x_ref, p_ref, w_ref, b_ref, o_ref):
    agg = jnp.concatenate([p_ref[0], p_ref[1]], axis=1)
    h = x_ref[...] + agg
    y = jnp.dot(h, w_ref[...], preferred_element_type=jnp.float32)
    o_ref[...] = jnp.maximum(y + b_ref[...], 0.0)


@jax.jit
def _tc_mlp(x, parts, w, b):
    bn = 1264
    grid = (N_PAD // bn,)
    return pl.pallas_call(
        _mlp_body,
        grid=grid,
        in_specs=[
            pl.BlockSpec((bn, D), lambda i: (i, 0)),
            pl.BlockSpec((NC, bn, FH), lambda i: (0, i, 0)),
            pl.BlockSpec((D, D), lambda i: (0, 0)),
            pl.BlockSpec((1, D), lambda i: (0, 0)),
        ],
        out_specs=pl.BlockSpec((bn, D), lambda i: (i, 0)),
        out_shape=jax.ShapeDtypeStruct((N_PAD, D), jnp.float32),
    )(x, parts, w, b.reshape(1, D))


def kernel(x, edge_index, W1, b1, W2, b2):
    pad = EPT_PAD - EPT
    srcq = jnp.pad(edge_index[0].reshape(NS, EPT), ((0, 0), (0, pad)),
                   constant_values=N).reshape(NS, NQ, QEDGE)
    dstq = jnp.pad(edge_index[1].reshape(NS, EPT), ((0, 0), (0, pad)),
                   constant_values=N).reshape(NS, NQ, QCHUNK, CHUNK)
    x2 = jnp.pad(x, ((0, N_PAD - N), (0, 0)))

    p1 = _sc_agg(jnp.stack([x2[:, :FH], x2[:, FH:]]), srcq, dstq)
    h = _tc_mlp(x2, p1, W1, b1)
    p2 = _sc_agg(jnp.stack([h[:, :FH], h[:, FH:]]), srcq, dstq)
    out = _tc_mlp(h, p2, W2, b2)
    return out[:N]
